# baseline (speedup 1.0000x reference)
import functools

import jax
import jax.numpy as jnp
from jax import lax
from jax.experimental import pallas as pl
from jax.experimental.pallas import tpu as pltpu
from jax.experimental.pallas import tpu_sc as plsc

BATCH = 16384
HIST_LEN = 50
EMBED_DIM = 64
VOCAB = 1000000

_info = plsc.get_sparse_core_info()
NC, NS = _info.num_cores, _info.num_subcores
NW = NC * NS
BPW = BATCH // NW
CB = BPW // 128
NBUF = 5
NCHUNK = CB * HIST_LEN

RBLK = 256
NBLK = VOCAB // RBLK
TAIL = VOCAB - NBLK * RBLK
BLK_PW = (NBLK + NW - 1) // NW
RNB = 3


def _repack_body(wt_hbm, out_hbm, ibuf, obuf, tibuf, isems, osems):
    wid = lax.axis_index("s") * NC + lax.axis_index("c")
    iota = lax.iota(jnp.int32, 16)
    pks = [lax.bitwise_and(iota + k, 15) for k in range(16)]
    rv = [lax.shift_right_logical(iota + 16 * b2, 1) for b2 in range(RBLK // 16)]
    pb = [lax.shift_left(lax.bitwise_and(iota + 16 * b2, 1), 6)
          for b2 in range(RBLK // 16)]

    def load_descs(bl, r):
        return [
            pltpu.make_async_copy(
                wt_hbm.at[pl.ds(8 * e, 8), pl.ds(bl * RBLK, RBLK)],
                ibuf.at[r, pl.ds(8 * e, 8)], isems.at[r])
            for e in range(8)
        ]

    def load_start(bl, r):
        for c in load_descs(bl, r):
            c.start()

    def load_wait(bl, r):
        for c in load_descs(bl, r):
            c.wait()

    def storeblk(bl, r):
        return pltpu.make_async_copy(
            obuf.at[r], out_hbm.at[pl.ds(bl * (RBLK // 2), RBLK // 2)],
            osems.at[r])

    def transpose(r, nb2):
        def ablock(a, carry):
            for b2 in range(nb2):
                for k in range(16):
                    rowd = pks[k] + 16 * a
                    vals = plsc.load_gather(ibuf.at[r], [rowd, iota + 16 * b2])
                    plsc.store_scatter(obuf.at[r], [rv[b2], pb[b2] + rowd], vals)
            return carry
        lax.fori_loop(0, 4, ablock, 0)

    base = wid * BLK_PW
    lim = jnp.minimum(base + BLK_PW, NBLK)

    for r in range(RNB):
        @pl.when(base + r < lim)
        def _():
            load_start(base + r, r)

    nsteps = (BLK_PW + RNB - 1) // RNB

    def gstep(g, carry):
        for r in range(RNB):
            bl = base + g * RNB + r

            @pl.when(bl < lim)
            def _():
                load_wait(bl, r)

                @pl.when(g > 0)
                def _():
                    storeblk(bl, r).wait()

                transpose(r, RBLK // 16)
                storeblk(bl, r).start()

                nxt = bl + RNB
                @pl.when(nxt < lim)
                def _():
                    load_start(nxt, r)
        return carry

    lax.fori_loop(0, nsteps, gstep, 0)
    for r in range(RNB):
        @pl.when(base + r < lim)
        def _():
            storeblk(base, r).wait()

    @pl.when(wid == NW - 1)
    def _():
        pltpu.sync_copy(wt_hbm.at[:, pl.ds(NBLK * RBLK, TAIL)], tibuf)

        def tail_block(a, carry):
            for b2 in range(4):
                for k in range(16):
                    rowd = pks[k] + 16 * a
                    vals = plsc.load_gather(tibuf, [rowd, iota + 16 * b2])
                    plsc.store_scatter(obuf.at[0], [rv[b2], pb[b2] + rowd], vals)
            return carry
        lax.fori_loop(0, 4, tail_block, 0)
        pltpu.sync_copy(obuf.at[0, pl.ds(0, TAIL // 2)],
                        out_hbm.at[pl.ds(NBLK * (RBLK // 2), TAIL // 2)])


def _embed_body(x_hbm, w_hbm, out_hbm, xs, cidx, gbuf, tbuf, gsems, ssems):
    wid = lax.axis_index("s") * NC + lax.axis_index("c")
    pltpu.sync_copy(
        x_hbm.at[pl.ds(pl.multiple_of(wid * (BPW * HIST_LEN), 8), BPW * HIST_LEN)],
        xs)

    iota = lax.iota(jnp.int32, 16)
    pks = [lax.bitwise_and(iota + k, 15) for k in range(16)]
    e_of = [lax.shift_right_logical(iota + 16 * u, 3) for u in range(4)]
    f_of = lax.bitwise_and(iota, 7)
    cols_u = [iota + 16 * u for u in range(4)]

    def build_cidx(t, cb, s):
        for v in range(8):
            flat = (iota + (128 * cb + 16 * v)) * HIST_LEN + t
            cidx[s, pl.ds(16 * v, 16)] = plsc.load_gather(xs, [flat])

    def gather_descs(s):
        return [
            pltpu.make_async_copy(
                w_hbm.at[cidx.at[s, pl.ds(0, 64)]],
                gbuf.at[s, pl.ds(0, 64)], gsems.at[s]),
            pltpu.make_async_copy(
                w_hbm.at[cidx.at[s, pl.ds(64, 64)]],
                gbuf.at[s, pl.ds(64, 64)], gsems.at[s]),
        ]

    def gather_start(s):
        for c in gather_descs(s):
            c.start()

    def gather_wait(s):
        for c in gather_descs(s):
            c.wait()

    def store(t, cb, s):
        cbg = wid * CB + cb
        return pltpu.make_async_copy(tbuf.at[s], out_hbm.at[t, :, cbg], ssems.at[s])

    def transpose(s):
        def vblock(v, carry):
            for k in range(16):
                rowm = pks[k] + 16 * v
                for u in range(4):
                    vals = plsc.load_gather(gbuf.at[s], [rowm, cols_u[u]])
                    plsc.store_scatter(tbuf.at[s], [e_of[u], f_of, rowm], vals)
            return carry
        lax.fori_loop(0, 8, vblock, 0)

    for k in range(NBUF):
        build_cidx(k // CB, k % CB, k)
        gather_start(k)

    def j_step(j, carry):
        for s in range(NBUF):
            k = j * NBUF + s
            t = lax.div(k, CB)
            cb = lax.rem(k, CB)
            gather_wait(s)

            @pl.when(k >= NBUF)
            def _():
                store(t, cb, s).wait()

            transpose(s)
            store(t, cb, s).start()

            kn = k + NBUF
            @pl.when(kn < NCHUNK)
            def _():
                build_cidx(lax.div(kn, CB), lax.rem(kn, CB), s)
                gather_start(s)
        return carry

    lax.fori_loop(0, NCHUNK // NBUF, j_step, 0)
    for k in range(NCHUNK - NBUF, NCHUNK):
        store(k // CB, k % CB, k % NBUF).wait()


@jax.jit
def _run(x_flat, wt):
    mesh = plsc.VectorSubcoreMesh(core_axis_name="c", subcore_axis_name="s")
    packed = pl.kernel(
        _repack_body,
        mesh=mesh,
        out_type=jax.ShapeDtypeStruct((VOCAB // 2, 2 * EMBED_DIM), jnp.float32),
        scratch_types=[
            pltpu.VMEM((RNB, EMBED_DIM, RBLK), jnp.float32),
            pltpu.VMEM((RNB, RBLK // 2, 128), jnp.float32),
            pltpu.VMEM((EMBED_DIM, TAIL), jnp.float32),
            pltpu.SemaphoreType.DMA((RNB,)),
            pltpu.SemaphoreType.DMA((RNB,)),
        ],
        compiler_params=pltpu.CompilerParams(
            use_tc_tiling_on_sc=True, needs_layout_passes=False),
    )(wt)
    w_lin = packed.reshape(VOCAB, EMBED_DIM)
    return pl.kernel(
        _embed_body,
        mesh=mesh,
        out_type=jax.ShapeDtypeStruct(
            (HIST_LEN, 8, BATCH // 128, 8, 128), jnp.float32),
        scratch_types=[
            pltpu.VMEM((BPW * HIST_LEN,), jnp.int32),
            pltpu.VMEM((NBUF, 128), jnp.int32),
            pltpu.VMEM((NBUF, 128, EMBED_DIM), jnp.float32),
            pltpu.VMEM((NBUF, 8, 8, 128), jnp.float32),
            pltpu.SemaphoreType.DMA((NBUF,)),
            pltpu.SemaphoreType.DMA((NBUF,)),
        ],
        compiler_params=pltpu.CompilerParams(
            use_tc_tiling_on_sc=False, needs_layout_passes=False),
    )(x_flat, w_lin)


def kernel(x, weight):
    p = _run(x.reshape(BATCH * HIST_LEN).astype(jnp.int32),
             jnp.transpose(weight))
    return p.transpose(2, 4, 0, 1, 3).reshape(BATCH, HIST_LEN, EMBED_DIM)


# speedup vs baseline: 1.1333x; 1.1333x over previous
import functools

import jax
import jax.numpy as jnp
from jax import lax
from jax.experimental import pallas as pl
from jax.experimental.pallas import tpu as pltpu
from jax.experimental.pallas import tpu_sc as plsc

BATCH = 16384
HIST_LEN = 50
EMBED_DIM = 64
VOCAB = 1000000

_info = plsc.get_sparse_core_info()
NC, NS = _info.num_cores, _info.num_subcores
NW = NC * NS
BPW = BATCH // NW
CB = BPW // 128
NBUF = CB


def _body(x_hbm, w_hbm, out_hbm, xs, cidx, gbuf, tbuf, gsems, ssems):
    wid = lax.axis_index("s") * NC + lax.axis_index("c")
    pltpu.sync_copy(
        x_hbm.at[pl.ds(pl.multiple_of(wid * (BPW * HIST_LEN), 8), BPW * HIST_LEN)],
        xs)

    iota = lax.iota(jnp.int32, 16)
    pks = [lax.bitwise_and(iota + k, 15) for k in range(16)]
    e_of = [lax.shift_right_logical(iota + 16 * u, 3) for u in range(4)]
    f_of = lax.bitwise_and(iota, 7)
    cols_u = [iota + 16 * u for u in range(4)]

    def build_cidx(t, b):
        for v in range(8):
            flat = (iota + (128 * b + 16 * v)) * HIST_LEN + t
            cidx[b, pl.ds(16 * v, 16)] = plsc.load_gather(xs, [flat])

    def gather(b):
        return pltpu.make_async_copy(w_hbm.at[cidx.at[b]], gbuf.at[b], gsems.at[b])

    def store(t, b):
        cbg = wid * CB + b
        return pltpu.make_async_copy(tbuf.at[b], out_hbm.at[t, :, cbg], ssems.at[b])

    def transpose(b):
        def vblock(v, carry):
            for k in range(16):
                rowm = pks[k] + 16 * v
                for u in range(4):
                    vals = plsc.load_gather(gbuf.at[b], [rowm, cols_u[u]])
                    plsc.store_scatter(tbuf.at[b], [e_of[u], f_of, rowm], vals)
            return carry
        lax.fori_loop(0, 8, vblock, 0)

    for b in range(NBUF):
        build_cidx(0, b)
        gather(b).start()

    def t_step(t, carry):
        for b in range(NBUF):
            gather(b).wait()

            @pl.when(t > 0)
            def _():
                store(t - 1, b).wait()

            transpose(b)
            store(t, b).start()

            @pl.when(t + 1 < HIST_LEN)
            def _():
                build_cidx(t + 1, b)
                gather(b).start()
        return carry

    lax.fori_loop(0, HIST_LEN, t_step, 0)
    for b in range(NBUF):
        store(HIST_LEN - 1, b).wait()


@jax.jit
def _embed(x_flat, w):
    mesh = plsc.VectorSubcoreMesh(core_axis_name="c", subcore_axis_name="s")
    return pl.kernel(
        _body,
        mesh=mesh,
        out_type=jax.ShapeDtypeStruct(
            (HIST_LEN, 8, BATCH // 128, 8, 128), jnp.float32),
        scratch_types=[
            pltpu.VMEM((BPW * HIST_LEN,), jnp.int32),
            pltpu.VMEM((NBUF, 128), jnp.int32),
            pltpu.VMEM((NBUF, 128, EMBED_DIM), jnp.float32),
            pltpu.VMEM((NBUF, 8, 8, 128), jnp.float32),
            pltpu.SemaphoreType.DMA((NBUF,)),
            pltpu.SemaphoreType.DMA((NBUF,)),
        ],
        compiler_params=pltpu.CompilerParams(
            use_tc_tiling_on_sc=False, needs_layout_passes=False),
    )(x_flat, w)


def kernel(x, weight):
    p = _embed(x.reshape(BATCH * HIST_LEN).astype(jnp.int32), weight)
    return p.transpose(2, 4, 0, 1, 3).reshape(BATCH, HIST_LEN, EMBED_DIM)
